# baseline (speedup 1.0000x reference)
import jax
import jax.numpy as jnp
import numpy as np
from jax.experimental import pallas as pl
from jax.experimental.pallas import tpu as pltpu

_ANCH = np.array(
    [[10, 13], [16, 30], [33, 23], [30, 61], [62, 45], [59, 119],
     [116, 90], [156, 198], [373, 326]], dtype=np.float32)
_GRIDW = (52, 26, 13)
_OFFS = (0, 8112, 10140)
_B = 8
_NT = 20


def _sel9(idx, vals):
    out = jnp.full(idx.shape, vals[8], dtype=jnp.float32)
    for k in range(7, -1, -1):
        out = jnp.where(idx == k, jnp.float32(vals[k]), out)
    return out


def _fused_body(misc_ref, tgs_ref, sl_ref, sm_ref, sh_ref, tgt_ref,
                lraw_ref, mraw_ref, hraw_ref, out_ref,
                gatl_ref, gatm_ref, gath_ref, sem_ref):
    b = pl.program_id(0)
    iw = misc_ref[0]
    tgt = tgt_ref[0]
    x1 = tgt[:, 0:1]
    y1 = tgt[:, 1:2]
    x2 = tgt[:, 2:3]
    y2 = tgt[:, 3:4]
    cls = tgt[:, 4:5]
    w_n = x2 - x1
    h_n = y2 - y1
    vld = (w_n > 0) & (h_n > 0)
    cxn = (x1 + x2) * 0.5
    cyn = (y1 + y2) * 0.5
    w_px = w_n * iw
    h_px = h_n * iw

    ai = jax.lax.broadcasted_iota(jnp.int32, (_NT, 9), 1)
    aw9 = _sel9(ai, _ANCH[:, 0])
    ah9 = _sel9(ai, _ANCH[:, 1])
    ainter = jnp.minimum(w_px, aw9) * jnp.minimum(h_px, ah9)
    aiou = ainter / (w_px * h_px + aw9 * ah9 - ainter + 1e-9)
    mx = jnp.max(aiou, axis=1, keepdims=True)
    astar = jnp.clip(
        jnp.min(jnp.where(aiou == mx, ai, 99), axis=1, keepdims=True), 0, 8)
    s = astar // 3
    aloc = astar % 3
    gw = jnp.where(s == 0, _GRIDW[0], jnp.where(s == 1, _GRIDW[1], _GRIDW[2]))
    off = jnp.where(s == 0, _OFFS[0], jnp.where(s == 1, _OFFS[1], _OFFS[2]))
    gwf = gw.astype(jnp.float32)
    gi = jnp.clip((cxn * gwf).astype(jnp.int32), 0, gw - 1)
    gj = jnp.clip((cyn * gwf).astype(jnp.int32), 0, gw - 1)
    n = off + (gj * gw + gi) * 3 + aloc

    n_s, vld_s, aloc_s, gi_s, gj_s, s_s = [], [], [], [], [], []
    for t in range(_NT):
        sx1 = tgs_ref[b, t, 0]
        sy1 = tgs_ref[b, t, 1]
        sx2 = tgs_ref[b, t, 2]
        sy2 = tgs_ref[b, t, 3]
        swn = sx2 - sx1
        shn = sy2 - sy1
        swp = swn * iw
        shp = shn * iw
        scx = (sx1 + sx2) * 0.5
        scy = (sy1 + sy2) * 0.5
        best = jnp.float32(-jnp.inf)
        asel = jnp.int32(99)
        for k in range(9):
            akw = float(_ANCH[k, 0])
            akh = float(_ANCH[k, 1])
            sint = jnp.minimum(swp, akw) * jnp.minimum(shp, akh)
            sio = sint / (swp * shp + akw * akh - sint + 1e-9)
            take = sio > best
            asel = jnp.where(take, k, asel)
            best = jnp.maximum(best, sio)
        asel = jnp.clip(asel, 0, 8)
        ss = asel // 3
        sal = asel % 3
        sgw = jnp.where(ss == 0, _GRIDW[0],
                        jnp.where(ss == 1, _GRIDW[1], _GRIDW[2]))
        soff = jnp.where(ss == 0, _OFFS[0],
                         jnp.where(ss == 1, _OFFS[1], _OFFS[2]))
        sgi = jnp.clip((scx * sgw.astype(jnp.float32)).astype(jnp.int32),
                       0, sgw - 1)
        sgj = jnp.clip((scy * sgw.astype(jnp.float32)).astype(jnp.int32),
                       0, sgw - 1)
        n_s.append(soff + (sgj * sgw + sgi) * 3 + sal)
        vld_s.append((swn > 0) & (shn > 0))
        aloc_s.append(sal)
        gi_s.append(sgi)
        gj_s.append(sgj)
        s_s.append(ss)

    raws = (lraw_ref, mraw_ref, hraw_ref)
    gats = (gatl_ref, gatm_ref, gath_ref)

    def _copy(lv, t):
        W = _GRIDW[lv]
        ch0 = aloc_s[t] * 85
        gjc = jnp.minimum(gj_s[t], W - 1)
        return pltpu.make_async_copy(
            raws[lv].at[b, pl.ds(ch0, 85), gjc],
            gats[lv].at[t],
            sem_ref.at[lv, t])

    for t in range(_NT):
        for lv in range(3):
            @pl.when(s_s[t] == lv)
            def _start(lv=lv, t=t):
                _copy(lv, t).start()

    back_sum = jnp.float32(0.0)
    for level, ref in ((0, sl_ref), (1, sm_ref), (2, sh_ref)):
        W = _GRIDW[level]
        OFF = _OFFS[level]
        stride = misc_ref[1 + level]
        gxi = jax.lax.broadcasted_iota(jnp.int32, (W, W), 1)
        gyi = jax.lax.broadcasted_iota(jnp.int32, (W, W), 0)
        gxf = gxi.astype(jnp.float32)
        gyf = gyi.astype(jnp.float32)
        nbase = OFF + (gyi * W + gxi) * 3
        for a in range(3):
            txs = jax.nn.sigmoid(ref[0, 5 * a + 0])
            tys = jax.nn.sigmoid(ref[0, 5 * a + 1])
            tw = ref[0, 5 * a + 2]
            th = ref[0, 5 * a + 3]
            conf_logit = ref[0, 5 * a + 4]
            cx = (txs + gxf) * stride
            cy = (tys + gyf) * stride
            aw = float(_ANCH[3 * level + a, 0])
            ah = float(_ANCH[3 * level + a, 1])
            bw = aw * jnp.exp(jnp.clip(tw, -10.0, 10.0))
            bh = ah * jnp.exp(jnp.clip(th, -10.0, 10.0))
            bx1 = cx - bw * 0.5
            by1 = cy - bh * 0.5
            bx2 = cx + bw * 0.5
            by2 = cy + bh * 0.5
            area_b = (bx2 - bx1) * (by2 - by1)
            max_iou = jnp.full((W, W), -1.0, jnp.float32)
            for t in range(_NT):
                gx1 = tgs_ref[b, t, 0] * iw
                gy1 = tgs_ref[b, t, 1] * iw
                gx2 = tgs_ref[b, t, 2] * iw
                gy2 = tgs_ref[b, t, 3] * iw
                area_g = (gx2 - gx1) * (gy2 - gy1)
                ix1 = jnp.maximum(bx1, gx1)
                iy1 = jnp.maximum(by1, gy1)
                ix2 = jnp.minimum(bx2, gx2)
                iy2 = jnp.minimum(by2, gy2)
                inter = (jnp.maximum(ix2 - ix1, 0.0)
                         * jnp.maximum(iy2 - iy1, 0.0))
                iou = inter / (area_b + area_g - inter + 1e-9)
                max_iou = jnp.maximum(max_iou, iou)
            back0 = max_iou <= 0.5
            n_glob = nbase + a
            fore = jnp.zeros((W, W), jnp.bool_)
            for t in range(_NT):
                fore = fore | ((n_glob == n_s[t]) & vld_s[t])
            conf = jnp.clip(jax.nn.sigmoid(conf_logit), 1e-7, 1.0 - 1e-7)
            term = jnp.where(back0 & jnp.logical_not(fore),
                             -jnp.log(1.0 - conf), 0.0)
            back_sum = back_sum + jnp.sum(term)

    for t in range(_NT):
        for lv in range(3):
            @pl.when(s_s[t] == lv)
            def _wait(lv=lv, t=t):
                _copy(lv, t).wait()
    rows = []
    for t in range(_NT):
        col = jnp.zeros((85, 1), jnp.float32)
        for lv in range(3):
            W = _GRIDW[lv]
            li = jax.lax.broadcasted_iota(jnp.int32, (1, W), 1)
            msk = (li == gi_s[t]) & (s_s[t] == lv)
            sel = jnp.where(msk, gats[lv][t], 0.0)
            col = col + jnp.sum(sel, axis=1, keepdims=True)
        rows.append(col.T)
    comp = jnp.concatenate(rows, axis=0)

    winner = jnp.full((_NT, 1), -1, jnp.int32)
    for tp in range(_NT):
        winner = jnp.where(vld[tp:tp + 1, :] & (n == n[tp:tp + 1, :]),
                           tp, winner)
    t_iota = jax.lax.broadcasted_iota(jnp.int32, (_NT, 1), 0)
    actf = (vld & (winner == t_iota)).astype(jnp.float32)

    awm = _sel9(astar, _ANCH[:, 0])
    ahm = _sel9(astar, _ANCH[:, 1])
    tx = cxn * gwf - gi.astype(jnp.float32)
    ty = cyn * gwf - gj.astype(jnp.float32)
    twt = jnp.log(jnp.maximum(w_px, 1.0) / awm)
    tht = jnp.log(jnp.maximum(h_px, 1.0) / ahm)
    scale = 2.0 - w_n * h_n

    sig0 = jax.nn.sigmoid(comp)
    px = sig0[:, 0:1]
    py = sig0[:, 1:2]
    pw = comp[:, 2:3]
    ph = comp[:, 3:4]
    pc = sig0[:, 4:5]
    sf = scale * actf
    xy_loss = jnp.sum(sf * ((px - tx) ** 2 + (py - ty) ** 2)) * 0.5
    wh_loss = jnp.sum(sf * ((pw - twt) ** 2 + (ph - tht) ** 2)) * 0.5
    pcc = jnp.clip(pc, 1e-7, 1.0 - 1e-7)
    conf_fore = jnp.sum(actf * (-jnp.log(pcc)))
    c_iota = jax.lax.broadcasted_iota(jnp.int32, (_NT, 85), 1)
    clsp = jnp.clip(sig0, 1e-7, 1.0 - 1e-7)
    onehot = c_iota == cls.astype(jnp.int32) + 5
    chm = c_iota >= 5
    bce = -jnp.where(onehot, jnp.log(clsp), jnp.log(1.0 - clsp))
    cls_loss = jnp.sum(jnp.where(chm, bce, 0.0) * actf)

    partial = xy_loss + wh_loss + conf_fore + cls_loss + back_sum
    prev = jnp.where(b == 0, 0.0, out_ref[0, 0, 0])
    tot = prev + partial
    out_ref[0, 0, 0] = jnp.where(b == _B - 1, tot / _B, tot)


_INTERPRET = False


def kernel(l_data, m_data, h_data, targets, input_wh):
    iw_i = jnp.asarray(input_wh)
    iw_f = iw_i.astype(jnp.float32)
    strides = [(iw_i // w).astype(jnp.float32) for w in _GRIDW]
    misc = jnp.stack([iw_f] + strides)
    sls = []
    for d in (l_data, m_data, h_data):
        sls.append(jnp.concatenate(
            [d[:, 85 * a:85 * a + 5] for a in range(3)], axis=1))
    out = pl.pallas_call(
        _fused_body,
        grid=(_B,),
        in_specs=[
            pl.BlockSpec(memory_space=pltpu.SMEM),
            pl.BlockSpec(memory_space=pltpu.SMEM),
            pl.BlockSpec((1, 15, _GRIDW[0], _GRIDW[0]),
                         lambda b: (b, 0, 0, 0)),
            pl.BlockSpec((1, 15, _GRIDW[1], _GRIDW[1]),
                         lambda b: (b, 0, 0, 0)),
            pl.BlockSpec((1, 15, _GRIDW[2], _GRIDW[2]),
                         lambda b: (b, 0, 0, 0)),
            pl.BlockSpec((1, _NT, 5), lambda b: (b, 0, 0)),
            pl.BlockSpec(memory_space=pl.ANY),
            pl.BlockSpec(memory_space=pl.ANY),
            pl.BlockSpec(memory_space=pl.ANY),
        ],
        out_specs=pl.BlockSpec((1, 1, 1), lambda b: (0, 0, 0),
                               memory_space=pltpu.SMEM),
        out_shape=jax.ShapeDtypeStruct((1, 1, 1), jnp.float32),
        scratch_shapes=[
            pltpu.VMEM((_NT, 85, _GRIDW[0]), jnp.float32),
            pltpu.VMEM((_NT, 85, _GRIDW[1]), jnp.float32),
            pltpu.VMEM((_NT, 85, _GRIDW[2]), jnp.float32),
            pltpu.SemaphoreType.DMA((3, _NT)),
        ],
        interpret=_INTERPRET,
    )(misc, targets, sls[0], sls[1], sls[2], targets,
      l_data, m_data, h_data)
    return out[0, 0, 0]
